# baseline (speedup 1.0000x reference)
import functools

import jax
import jax.numpy as jnp
from jax import lax
from jax.experimental import pallas as pl
from jax.experimental.pallas import tpu as pltpu
from jax.experimental.pallas import tpu_sc as plsc

B = 4
N_DUAL = 81920
N_VERTEX = 40962
D = 64
NU = 3

NC = 2
NS = 16
NW = NC * NS

ROWS = B * N_VERTEX * NU
CHUNK = 128
CPT = 120
REM = ROWS - NW * CPT * CHUNK
IDX_LOAD = CPT + 8
IDX_ROWS = NW * CPT + 8

_mesh = plsc.VectorSubcoreMesh(core_axis_name="c", subcore_axis_name="s")


@functools.partial(
    pl.kernel,
    out_type=jax.ShapeDtypeStruct((ROWS, D), jnp.float32),
    mesh=_mesh,
    scratch_types=[
        pltpu.VMEM((IDX_LOAD, CHUNK), jnp.int32),
        pltpu.VMEM((CHUNK, D), jnp.float32),
        pltpu.SemaphoreType.DMA,
    ],
    compiler_params=pltpu.CompilerParams(use_tc_tiling_on_sc=False),
)
def _gather_kernel(z_hbm, idx_hbm, out_hbm, idx_v, buf_v, gsem):
    c = lax.axis_index("c")
    s = lax.axis_index("s")
    wid = s * NC + c

    pltpu.sync_copy(idx_hbm.at[pl.ds(wid * CPT, IDX_LOAD)], idx_v)

    def step(j, carry):
        pltpu.async_copy(z_hbm.at[idx_v.at[j]], buf_v, gsem).wait()
        pltpu.sync_copy(buf_v, out_hbm.at[pl.ds((wid * CPT + j) * CHUNK, CHUNK)])
        return carry

    lax.fori_loop(0, CPT, step, 0)

    @pl.when(wid == NW - 1)
    def _remainder():
        pltpu.async_copy(z_hbm.at[idx_v.at[CPT]], buf_v, gsem).wait()
        pltpu.sync_copy(buf_v.at[pl.ds(0, REM)],
                        out_hbm.at[pl.ds(NW * CPT * CHUNK, REM)])


def kernel(z_prime, x_ancil, index):
    del x_ancil
    idx = index.astype(jnp.int32).reshape(-1)
    idx_flat = (idx[None, :]
                + (jnp.arange(B, dtype=jnp.int32) * N_DUAL)[:, None]).reshape(-1)
    idx_flat = jnp.pad(idx_flat, (0, IDX_ROWS * CHUNK - ROWS))
    idx2 = idx_flat.reshape(IDX_ROWS, CHUNK)
    z_flat = z_prime.reshape(B * N_DUAL, D)
    out = _gather_kernel(z_flat, idx2)
    return out.reshape(B, N_VERTEX, NU * D)


# speedup vs baseline: 1.0727x; 1.0727x over previous
import functools

import jax
import jax.numpy as jnp
from jax import lax
from jax.experimental import pallas as pl
from jax.experimental.pallas import tpu as pltpu
from jax.experimental.pallas import tpu_sc as plsc

B = 4
N_DUAL = 81920
N_VERTEX = 40962
D = 64
NU = 3

NC = 2
NS = 16
NW = NC * NS

ROWS = B * N_VERTEX * NU
CHUNK = 128
CPT = 120
REM = ROWS - NW * CPT * CHUNK
IDX_LOAD = CPT + 8
IDX_ROWS = NW * CPT + 8

_mesh = plsc.VectorSubcoreMesh(core_axis_name="c", subcore_axis_name="s")


NBUF = 8
NGROUP = CPT // NBUF


@functools.partial(
    pl.kernel,
    out_type=jax.ShapeDtypeStruct((ROWS, D), jnp.float32),
    mesh=_mesh,
    scratch_types=[
        pltpu.VMEM((IDX_LOAD, CHUNK), jnp.int32),
        pltpu.VMEM((NBUF, CHUNK, D), jnp.float32),
    ]
    + [pltpu.SemaphoreType.DMA] * (2 * NBUF),
    compiler_params=pltpu.CompilerParams(use_tc_tiling_on_sc=False),
)
def _gather_kernel(z_hbm, idx_hbm, out_hbm, idx_v, buf_v, *sems):
    gsem = sems[:NBUF]
    osem = sems[NBUF:]
    c = lax.axis_index("c")
    s = lax.axis_index("s")
    wid = s * NC + c
    base = wid * CPT

    pltpu.sync_copy(idx_hbm.at[pl.ds(wid * CPT, IDX_LOAD)], idx_v)

    def start_gather(j, slot):
        pltpu.async_copy(z_hbm.at[idx_v.at[j]], buf_v.at[slot], gsem[slot])

    def start_out(j, slot):
        pltpu.async_copy(buf_v.at[slot],
                         out_hbm.at[pl.ds((base + j) * CHUNK, CHUNK)],
                         osem[slot])

    for slot in range(NBUF):
        start_gather(slot, slot)

    def group(g, carry):
        jg = g * NBUF
        for slot in range(NBUF):
            pltpu.make_async_copy(buf_v.at[slot],
                                  out_hbm.at[pl.ds(0, CHUNK)],
                                  gsem[slot]).wait()
            start_out(jg + slot, slot)
        for slot in range(NBUF):
            pltpu.make_async_copy(buf_v.at[slot],
                                  out_hbm.at[pl.ds(0, CHUNK)],
                                  osem[slot]).wait()
            start_gather(jg + NBUF + slot, slot)
        return carry

    lax.fori_loop(0, NGROUP - 1, group, 0)

    jg = (NGROUP - 1) * NBUF
    for slot in range(NBUF):
        pltpu.make_async_copy(buf_v.at[slot],
                              out_hbm.at[pl.ds(0, CHUNK)],
                              gsem[slot]).wait()
        start_out(jg + slot, slot)
    for slot in range(NBUF):
        pltpu.make_async_copy(buf_v.at[slot],
                              out_hbm.at[pl.ds(0, CHUNK)],
                              osem[slot]).wait()

    @pl.when(wid == NW - 1)
    def _remainder():
        pltpu.async_copy(z_hbm.at[idx_v.at[CPT]], buf_v.at[0], gsem[0]).wait()
        pltpu.sync_copy(buf_v.at[0].at[pl.ds(0, REM)],
                        out_hbm.at[pl.ds(NW * CPT * CHUNK, REM)])


def kernel(z_prime, x_ancil, index):
    del x_ancil
    idx = index.astype(jnp.int32).reshape(-1)
    idx_flat = (idx[None, :]
                + (jnp.arange(B, dtype=jnp.int32) * N_DUAL)[:, None]).reshape(-1)
    idx_flat = jnp.pad(idx_flat, (0, IDX_ROWS * CHUNK - ROWS))
    idx2 = idx_flat.reshape(IDX_ROWS, CHUNK)
    z_flat = z_prime.reshape(B * N_DUAL, D)
    out = _gather_kernel(z_flat, idx2)
    return out.reshape(B, N_VERTEX, NU * D)
